# baseline (device time: 164020 ns/iter reference)
import jax
import jax.numpy as jnp
from jax import lax
from jax.experimental import pallas as pl
from jax.experimental.pallas import tpu as pltpu

N_DEV = 32
SQ = 1024
HQ = 8
DH = 128
DM = HQ * DH
CHUNK = SQ // N_DEV
PAY = DM + 2 * HQ
SCALE = 0.08838834764831843
NEG = -1e9


def kernel(x, Wq, K_ext, V_ext, Wo):
    skv = K_ext.shape[1]

    def body(x_ref, wq_ref, k_ref, v_ref, wo_ref, out_ref,
             sbuf, rs, send_rs, recv_rs, send_ag, recv_ag):
        my = lax.axis_index("i")

        xw = jnp.dot(x_ref[0], wq_ref[...],
                     preferred_element_type=jnp.float32)
        kv_k = k_ref[0]
        kv_v = v_ref[0]
        ri = lax.broadcasted_iota(jnp.int32, (SQ, skv), 0)
        ci = lax.broadcasted_iota(jnp.int32, (SQ, skv), 1)
        mask = ((ri // 64) % 4) == ((ci // 64) % 4)
        ms = []
        ls = []
        for h in range(HQ):
            qh = xw[:, h * DH:(h + 1) * DH]
            s = lax.dot_general(qh, kv_k[:, h, :], (((1,), (1,)), ((), ())),
                                preferred_element_type=jnp.float32)
            s = jnp.where(mask, s * SCALE, NEG)
            m = jnp.max(s, axis=1)
            w = jnp.exp(s - m[:, None])
            ls.append(jnp.sum(w, axis=1))
            ms.append(m)
            sbuf[:, h * DH:(h + 1) * DH] = jnp.dot(
                w, kv_v[:, h, :], preferred_element_type=jnp.float32)
        sbuf[:, DM:DM + HQ] = jnp.stack(ms, axis=1)
        sbuf[:, DM + HQ:PAY] = jnp.stack(ls, axis=1)

        for j in range(N_DEV):
            pltpu.make_async_remote_copy(
                src_ref=sbuf.at[pl.ds(j * CHUNK, CHUNK), :],
                dst_ref=rs.at[my],
                send_sem=send_rs.at[j],
                recv_sem=recv_rs.at[my],
                device_id=(j,),
                device_id_type=pl.DeviceIdType.MESH,
            ).start()
        for j in range(N_DEV):
            pltpu.make_async_remote_copy(
                src_ref=rs.at[j], dst_ref=rs.at[j],
                send_sem=recv_rs.at[j], recv_sem=recv_rs.at[j],
                device_id=(j,), device_id_type=pl.DeviceIdType.MESH,
            ).wait_recv()

        parts = rs[...]
        n_p = parts[:, :, :DM]
        m_p = parts[:, :, DM:DM + HQ]
        l_p = parts[:, :, DM + HQ:PAY]
        m_g = jnp.max(m_p, axis=0)
        sc = jnp.exp(m_p - m_g[None])
        l_g = jnp.sum(l_p * sc, axis=0)
        sc_full = jnp.broadcast_to(
            sc[..., None], (N_DEV, CHUNK, HQ, DH)).reshape(N_DEV, CHUNK, DM)
        n_g = jnp.sum(n_p * sc_full, axis=0)
        l_full = jnp.broadcast_to(
            l_g[..., None], (CHUNK, HQ, DH)).reshape(CHUNK, DM)
        ctx = n_g / l_full
        out_ref[0, pl.ds(my * CHUNK, CHUNK), :] = jnp.dot(
            ctx, wo_ref[...], preferred_element_type=jnp.float32)

        for j in range(N_DEV):
            @pl.when(j != my)
            def _():
                pltpu.make_async_remote_copy(
                    src_ref=out_ref.at[0, pl.ds(my * CHUNK, CHUNK), :],
                    dst_ref=out_ref.at[0, pl.ds(my * CHUNK, CHUNK), :],
                    send_sem=send_ag.at[j],
                    recv_sem=recv_ag.at[my],
                    device_id=(j,),
                    device_id_type=pl.DeviceIdType.MESH,
                ).start()
        for j in range(N_DEV):
            @pl.when(j != my)
            def _():
                pltpu.make_async_remote_copy(
                    src_ref=out_ref.at[0, pl.ds(j * CHUNK, CHUNK), :],
                    dst_ref=out_ref.at[0, pl.ds(j * CHUNK, CHUNK), :],
                    send_sem=recv_ag.at[j], recv_sem=recv_ag.at[j],
                    device_id=(j,), device_id_type=pl.DeviceIdType.MESH,
                ).wait_recv()

        for j in range(N_DEV):
            pltpu.make_async_remote_copy(
                src_ref=sbuf.at[pl.ds(j * CHUNK, CHUNK), :],
                dst_ref=rs.at[my],
                send_sem=send_rs.at[j], recv_sem=recv_rs.at[my],
                device_id=(j,), device_id_type=pl.DeviceIdType.MESH,
            ).wait_send()
            @pl.when(j != my)
            def _():
                pltpu.make_async_remote_copy(
                    src_ref=out_ref.at[0, pl.ds(my * CHUNK, CHUNK), :],
                    dst_ref=out_ref.at[0, pl.ds(my * CHUNK, CHUNK), :],
                    send_sem=send_ag.at[j], recv_sem=recv_ag.at[my],
                    device_id=(j,), device_id_type=pl.DeviceIdType.MESH,
                ).wait_send()

    return pl.pallas_call(
        body,
        out_shape=jax.ShapeDtypeStruct((1, SQ, DM), jnp.float32),
        in_specs=[pl.BlockSpec(memory_space=pltpu.VMEM)] * 5,
        out_specs=pl.BlockSpec(memory_space=pltpu.VMEM),
        scratch_shapes=[
            pltpu.VMEM((SQ, PAY), jnp.float32),
            pltpu.VMEM((N_DEV, CHUNK, PAY), jnp.float32),
            pltpu.SemaphoreType.DMA((N_DEV,)),
            pltpu.SemaphoreType.DMA((N_DEV,)),
            pltpu.SemaphoreType.DMA((N_DEV,)),
            pltpu.SemaphoreType.DMA((N_DEV,)),
        ],
        compiler_params=pltpu.CompilerParams(
            vmem_limit_bytes=128 * 1024 * 1024,
        ),
    )(x, Wq, K_ext, V_ext, Wo)


# device time: 155415 ns/iter; 1.0554x vs baseline; 1.0554x over previous
import jax
import jax.numpy as jnp
from jax import lax
from jax.experimental import pallas as pl
from jax.experimental.pallas import tpu as pltpu

N_DEV = 32
SQ = 1024
HQ = 8
DH = 128
DM = HQ * DH
CHUNK = SQ // N_DEV
PAY = DM + 2 * HQ
SCALE = 0.08838834764831843
NEG = -1e9


def kernel(x, Wq, K_ext, V_ext, Wo):
    skv = K_ext.shape[1]

    def body(x_ref, wq_ref, k_ref, v_ref, wo_ref, out_ref,
             sbuf, rs, send_rs, recv_rs, send_ag, recv_ag):
        my = lax.axis_index("i")

        xw = jnp.dot(x_ref[0], wq_ref[...],
                     preferred_element_type=jnp.float32)
        kv_k = k_ref[0]
        kv_v = v_ref[0]
        for c in range(4):
            blocks = [64 * (c + 4 * g) for g in range(4)]
            q_c = jnp.concatenate([xw[b:b + 64, :] for b in blocks], axis=0)
            k_c = jnp.concatenate([kv_k[b:b + 64] for b in blocks], axis=0)
            v_c = jnp.concatenate([kv_v[b:b + 64] for b in blocks], axis=0)
            ms = []
            ls = []
            for h in range(HQ):
                s = lax.dot_general(
                    q_c[:, h * DH:(h + 1) * DH], k_c[:, h, :],
                    (((1,), (1,)), ((), ())),
                    preferred_element_type=jnp.float32) * SCALE
                m = jnp.max(s, axis=1)
                w = jnp.exp(s - m[:, None])
                ls.append(jnp.sum(w, axis=1))
                ms.append(m)
                n = jnp.dot(w, v_c[:, h, :],
                            preferred_element_type=jnp.float32)
                for g in range(4):
                    sbuf[blocks[g]:blocks[g] + 64,
                         h * DH:(h + 1) * DH] = n[64 * g:64 * g + 64]
            m_st = jnp.stack(ms, axis=1)
            l_st = jnp.stack(ls, axis=1)
            for g in range(4):
                sbuf[blocks[g]:blocks[g] + 64,
                     DM:DM + HQ] = m_st[64 * g:64 * g + 64]
                sbuf[blocks[g]:blocks[g] + 64,
                     DM + HQ:PAY] = l_st[64 * g:64 * g + 64]

        for j in range(N_DEV):
            pltpu.make_async_remote_copy(
                src_ref=sbuf.at[pl.ds(j * CHUNK, CHUNK), :],
                dst_ref=rs.at[my],
                send_sem=send_rs.at[j],
                recv_sem=recv_rs.at[my],
                device_id=(j,),
                device_id_type=pl.DeviceIdType.MESH,
            ).start()
        for j in range(N_DEV):
            pltpu.make_async_remote_copy(
                src_ref=rs.at[j], dst_ref=rs.at[j],
                send_sem=recv_rs.at[j], recv_sem=recv_rs.at[j],
                device_id=(j,), device_id_type=pl.DeviceIdType.MESH,
            ).wait_recv()

        parts = rs[...]
        n_p = parts[:, :, :DM]
        m_p = parts[:, :, DM:DM + HQ]
        l_p = parts[:, :, DM + HQ:PAY]
        m_g = jnp.max(m_p, axis=0)
        sc = jnp.exp(m_p - m_g[None])
        l_g = jnp.sum(l_p * sc, axis=0)
        sc_full = jnp.broadcast_to(
            sc[..., None], (N_DEV, CHUNK, HQ, DH)).reshape(N_DEV, CHUNK, DM)
        n_g = jnp.sum(n_p * sc_full, axis=0)
        l_full = jnp.broadcast_to(
            l_g[..., None], (CHUNK, HQ, DH)).reshape(CHUNK, DM)
        ctx = n_g / l_full
        out_ref[0, pl.ds(my * CHUNK, CHUNK), :] = jnp.dot(
            ctx, wo_ref[...], preferred_element_type=jnp.float32)

        for j in range(N_DEV):
            @pl.when(j != my)
            def _():
                pltpu.make_async_remote_copy(
                    src_ref=out_ref.at[0, pl.ds(my * CHUNK, CHUNK), :],
                    dst_ref=out_ref.at[0, pl.ds(my * CHUNK, CHUNK), :],
                    send_sem=send_ag.at[j],
                    recv_sem=recv_ag.at[my],
                    device_id=(j,),
                    device_id_type=pl.DeviceIdType.MESH,
                ).start()
        for j in range(N_DEV):
            @pl.when(j != my)
            def _():
                pltpu.make_async_remote_copy(
                    src_ref=out_ref.at[0, pl.ds(j * CHUNK, CHUNK), :],
                    dst_ref=out_ref.at[0, pl.ds(j * CHUNK, CHUNK), :],
                    send_sem=recv_ag.at[j], recv_sem=recv_ag.at[j],
                    device_id=(j,), device_id_type=pl.DeviceIdType.MESH,
                ).wait_recv()

        for j in range(N_DEV):
            pltpu.make_async_remote_copy(
                src_ref=sbuf.at[pl.ds(j * CHUNK, CHUNK), :],
                dst_ref=rs.at[my],
                send_sem=send_rs.at[j], recv_sem=recv_rs.at[my],
                device_id=(j,), device_id_type=pl.DeviceIdType.MESH,
            ).wait_send()
            @pl.when(j != my)
            def _():
                pltpu.make_async_remote_copy(
                    src_ref=out_ref.at[0, pl.ds(my * CHUNK, CHUNK), :],
                    dst_ref=out_ref.at[0, pl.ds(my * CHUNK, CHUNK), :],
                    send_sem=send_ag.at[j], recv_sem=recv_ag.at[my],
                    device_id=(j,), device_id_type=pl.DeviceIdType.MESH,
                ).wait_send()

    return pl.pallas_call(
        body,
        out_shape=jax.ShapeDtypeStruct((1, SQ, DM), jnp.float32),
        in_specs=[pl.BlockSpec(memory_space=pltpu.VMEM)] * 5,
        out_specs=pl.BlockSpec(memory_space=pltpu.VMEM),
        scratch_shapes=[
            pltpu.VMEM((SQ, PAY), jnp.float32),
            pltpu.VMEM((N_DEV, CHUNK, PAY), jnp.float32),
            pltpu.SemaphoreType.DMA((N_DEV,)),
            pltpu.SemaphoreType.DMA((N_DEV,)),
            pltpu.SemaphoreType.DMA((N_DEV,)),
            pltpu.SemaphoreType.DMA((N_DEV,)),
        ],
        compiler_params=pltpu.CompilerParams(
            vmem_limit_bytes=128 * 1024 * 1024,
        ),
    )(x, Wq, K_ext, V_ext, Wo)


# device time: 100334 ns/iter; 1.6347x vs baseline; 1.5490x over previous
import jax
import jax.numpy as jnp
from jax import lax
from jax.experimental import pallas as pl
from jax.experimental.pallas import tpu as pltpu

N_DEV = 32
SQ = 1024
HQ = 8
DH = 128
DM = HQ * DH
CHUNK = SQ // N_DEV
ML = 2 * HQ
SCALE = 0.08838834764831843


def kernel(x, Wq, K_ext, V_ext, Wo):
    def body(x_ref, wq_ref, k_ref, v_ref, wo_ref, out_ref,
             nbuf, mlbuf, rs_n, rs_ml, agbuf,
             send_n, recv_n, send_ml, recv_ml, send_ag, recv_ag):
        my = lax.axis_index("i")

        xw = jnp.dot(x_ref[0], wq_ref[...],
                     preferred_element_type=jnp.float32)
        kv_k = k_ref[0]
        kv_v = v_ref[0]
        for c in range(4):
            blocks = [64 * (c + 4 * g) for g in range(4)]
            q_c = jnp.concatenate([xw[b:b + 64, :] for b in blocks], axis=0)
            k_c = jnp.concatenate([kv_k[b:b + 64] for b in blocks], axis=0)
            v_c = jnp.concatenate([kv_v[b:b + 64] for b in blocks], axis=0)
            ms = []
            ls = []
            for h in range(HQ):
                s = lax.dot_general(
                    q_c[:, h * DH:(h + 1) * DH], k_c[:, h, :],
                    (((1,), (1,)), ((), ())),
                    preferred_element_type=jnp.float32) * SCALE
                m = jnp.max(s, axis=1)
                w = jnp.exp(s - m[:, None])
                ls.append(jnp.sum(w, axis=1))
                ms.append(m)
                n = jnp.dot(w, v_c[:, h, :],
                            preferred_element_type=jnp.float32)
                for g in range(4):
                    nbuf[blocks[g]:blocks[g] + 64,
                         h * DH:(h + 1) * DH] = n[64 * g:64 * g + 64].astype(
                             jnp.bfloat16)
            m_st = jnp.stack(ms, axis=1)
            l_st = jnp.stack(ls, axis=1)
            for g in range(4):
                mlbuf[blocks[g]:blocks[g] + 64,
                      :HQ] = m_st[64 * g:64 * g + 64]
                mlbuf[blocks[g]:blocks[g] + 64,
                      HQ:ML] = l_st[64 * g:64 * g + 64]

        for j in range(N_DEV):
            pltpu.make_async_remote_copy(
                src_ref=nbuf.at[pl.ds(j * CHUNK, CHUNK), :],
                dst_ref=rs_n.at[my],
                send_sem=send_n.at[j],
                recv_sem=recv_n.at[my],
                device_id=(j,),
                device_id_type=pl.DeviceIdType.MESH,
            ).start()
            pltpu.make_async_remote_copy(
                src_ref=mlbuf.at[pl.ds(j * CHUNK, CHUNK), :],
                dst_ref=rs_ml.at[my],
                send_sem=send_ml.at[j],
                recv_sem=recv_ml.at[my],
                device_id=(j,),
                device_id_type=pl.DeviceIdType.MESH,
            ).start()
        for j in range(N_DEV):
            pltpu.make_async_remote_copy(
                src_ref=rs_n.at[j], dst_ref=rs_n.at[j],
                send_sem=recv_n.at[j], recv_sem=recv_n.at[j],
                device_id=(j,), device_id_type=pl.DeviceIdType.MESH,
            ).wait_recv()
            pltpu.make_async_remote_copy(
                src_ref=rs_ml.at[j], dst_ref=rs_ml.at[j],
                send_sem=recv_ml.at[j], recv_sem=recv_ml.at[j],
                device_id=(j,), device_id_type=pl.DeviceIdType.MESH,
            ).wait_recv()

        n_p = rs_n[...].astype(jnp.float32)
        m_p = rs_ml[:, :, :HQ]
        l_p = rs_ml[:, :, HQ:ML]
        m_g = jnp.max(m_p, axis=0)
        sc = jnp.exp(m_p - m_g[None])
        l_g = jnp.sum(l_p * sc, axis=0)
        sc_full = jnp.broadcast_to(
            sc[..., None], (N_DEV, CHUNK, HQ, DH)).reshape(N_DEV, CHUNK, DM)
        n_g = jnp.sum(n_p * sc_full, axis=0)
        l_full = jnp.broadcast_to(
            l_g[..., None], (CHUNK, HQ, DH)).reshape(CHUNK, DM)
        ctx = n_g / l_full
        out_chunk = jnp.dot(ctx, wo_ref[...],
                            preferred_element_type=jnp.float32)
        agbuf[pl.ds(my * CHUNK, CHUNK), :] = out_chunk.astype(jnp.bfloat16)

        for j in range(N_DEV):
            @pl.when(j != my)
            def _():
                pltpu.make_async_remote_copy(
                    src_ref=agbuf.at[pl.ds(my * CHUNK, CHUNK), :],
                    dst_ref=agbuf.at[pl.ds(my * CHUNK, CHUNK), :],
                    send_sem=send_ag.at[j],
                    recv_sem=recv_ag.at[my],
                    device_id=(j,),
                    device_id_type=pl.DeviceIdType.MESH,
                ).start()
        for j in range(N_DEV):
            @pl.when(j != my)
            def _():
                pltpu.make_async_remote_copy(
                    src_ref=agbuf.at[pl.ds(j * CHUNK, CHUNK), :],
                    dst_ref=agbuf.at[pl.ds(j * CHUNK, CHUNK), :],
                    send_sem=recv_ag.at[j], recv_sem=recv_ag.at[j],
                    device_id=(j,), device_id_type=pl.DeviceIdType.MESH,
                ).wait_recv()
        out_ref[0, :, :] = agbuf[...].astype(jnp.float32)

        for j in range(N_DEV):
            pltpu.make_async_remote_copy(
                src_ref=nbuf.at[pl.ds(j * CHUNK, CHUNK), :],
                dst_ref=rs_n.at[my],
                send_sem=send_n.at[j], recv_sem=recv_n.at[my],
                device_id=(j,), device_id_type=pl.DeviceIdType.MESH,
            ).wait_send()
            pltpu.make_async_remote_copy(
                src_ref=mlbuf.at[pl.ds(j * CHUNK, CHUNK), :],
                dst_ref=rs_ml.at[my],
                send_sem=send_ml.at[j], recv_sem=recv_ml.at[my],
                device_id=(j,), device_id_type=pl.DeviceIdType.MESH,
            ).wait_send()
            @pl.when(j != my)
            def _():
                pltpu.make_async_remote_copy(
                    src_ref=agbuf.at[pl.ds(my * CHUNK, CHUNK), :],
                    dst_ref=agbuf.at[pl.ds(my * CHUNK, CHUNK), :],
                    send_sem=send_ag.at[j], recv_sem=recv_ag.at[my],
                    device_id=(j,), device_id_type=pl.DeviceIdType.MESH,
                ).wait_send()

    return pl.pallas_call(
        body,
        out_shape=jax.ShapeDtypeStruct((1, SQ, DM), jnp.float32),
        in_specs=[pl.BlockSpec(memory_space=pltpu.VMEM)] * 5,
        out_specs=pl.BlockSpec(memory_space=pltpu.VMEM),
        scratch_shapes=[
            pltpu.VMEM((SQ, DM), jnp.bfloat16),
            pltpu.VMEM((SQ, ML), jnp.float32),
            pltpu.VMEM((N_DEV, CHUNK, DM), jnp.bfloat16),
            pltpu.VMEM((N_DEV, CHUNK, ML), jnp.float32),
            pltpu.VMEM((SQ, DM), jnp.bfloat16),
            pltpu.SemaphoreType.DMA((N_DEV,)),
            pltpu.SemaphoreType.DMA((N_DEV,)),
            pltpu.SemaphoreType.DMA((N_DEV,)),
            pltpu.SemaphoreType.DMA((N_DEV,)),
            pltpu.SemaphoreType.DMA((N_DEV,)),
            pltpu.SemaphoreType.DMA((N_DEV,)),
        ],
        compiler_params=pltpu.CompilerParams(
            vmem_limit_bytes=128 * 1024 * 1024,
        ),
    )(x, Wq, K_ext, V_ext, Wo)


# device time: 19145 ns/iter; 8.5672x vs baseline; 5.2407x over previous
import jax
import jax.numpy as jnp
from jax import lax
from jax.experimental import pallas as pl
from jax.experimental.pallas import tpu as pltpu

N_DEV = 32
SQ = 1024
HQ = 8
DH = 128
DM = HQ * DH
CHUNK = SQ // N_DEV
ML = 2 * HQ
SCALE = 0.08838834764831843


def kernel(x, Wq, K_ext, V_ext, Wo):
    def body(x_ref, wq_ref, k_ref, v_ref, wo_ref, out_ref,
             nbuf, mlbuf, rs_n, rs_ml, agbuf,
             send_n, recv_n, send_ml, recv_ml, send_ag, recv_ag):
        my = lax.axis_index("i")

        xw = jnp.dot(x_ref[0], wq_ref[...],
                     preferred_element_type=jnp.float32)
        kv_k = k_ref[0]
        kv_v = v_ref[0]
        for c in range(4):
            blocks = [64 * (c + 4 * g) for g in range(4)]
            q_c = jnp.concatenate([xw[b:b + 64, :] for b in blocks], axis=0)
            k_c = jnp.concatenate([kv_k[b:b + 64] for b in blocks], axis=0)
            v_c = jnp.concatenate([kv_v[b:b + 64] for b in blocks], axis=0)
            ms = []
            ls = []
            for h in range(HQ):
                s = lax.dot_general(
                    q_c[:, h * DH:(h + 1) * DH], k_c[:, h, :],
                    (((1,), (1,)), ((), ())),
                    preferred_element_type=jnp.float32) * SCALE
                m = jnp.max(s, axis=1)
                w = jnp.exp(s - m[:, None])
                ls.append(jnp.sum(w, axis=1))
                ms.append(m)
                n = jnp.dot(w, v_c[:, h, :],
                            preferred_element_type=jnp.float32)
                for g in range(4):
                    nbuf[blocks[g]:blocks[g] + 64,
                         h * DH:(h + 1) * DH] = n[64 * g:64 * g + 64].astype(
                             jnp.bfloat16)
            m_st = jnp.stack(ms, axis=1)
            l_st = jnp.stack(ls, axis=1)
            for g in range(4):
                mlbuf[blocks[g]:blocks[g] + 64,
                      :HQ] = m_st[64 * g:64 * g + 64]
                mlbuf[blocks[g]:blocks[g] + 64,
                      HQ:ML] = l_st[64 * g:64 * g + 64]
            for b in blocks:
                for j in (2 * b // 64, 2 * b // 64 + 1):
                    pltpu.make_async_remote_copy(
                        src_ref=nbuf.at[pl.ds(j * CHUNK, CHUNK), :],
                        dst_ref=rs_n.at[my],
                        send_sem=send_n.at[j],
                        recv_sem=recv_n.at[my],
                        device_id=(j,),
                        device_id_type=pl.DeviceIdType.MESH,
                    ).start()
                    pltpu.make_async_remote_copy(
                        src_ref=mlbuf.at[pl.ds(j * CHUNK, CHUNK), :],
                        dst_ref=rs_ml.at[my],
                        send_sem=send_ml.at[j],
                        recv_sem=recv_ml.at[my],
                        device_id=(j,),
                        device_id_type=pl.DeviceIdType.MESH,
                    ).start()

        for j in range(N_DEV):
            pltpu.make_async_remote_copy(
                src_ref=rs_n.at[j], dst_ref=rs_n.at[j],
                send_sem=recv_n.at[j], recv_sem=recv_n.at[j],
                device_id=(j,), device_id_type=pl.DeviceIdType.MESH,
            ).wait_recv()
            pltpu.make_async_remote_copy(
                src_ref=rs_ml.at[j], dst_ref=rs_ml.at[j],
                send_sem=recv_ml.at[j], recv_sem=recv_ml.at[j],
                device_id=(j,), device_id_type=pl.DeviceIdType.MESH,
            ).wait_recv()

        n_p = rs_n[...].astype(jnp.float32)
        m_p = rs_ml[:, :, :HQ]
        l_p = rs_ml[:, :, HQ:ML]
        m_g = jnp.max(m_p, axis=0)
        sc = jnp.exp(m_p - m_g[None])
        l_g = jnp.sum(l_p * sc, axis=0)
        sc_full = jnp.broadcast_to(
            sc[..., None], (N_DEV, CHUNK, HQ, DH)).reshape(N_DEV, CHUNK, DM)
        n_g = jnp.sum(n_p * sc_full, axis=0)
        l_full = jnp.broadcast_to(
            l_g[..., None], (CHUNK, HQ, DH)).reshape(CHUNK, DM)
        ctx = n_g / l_full
        out_chunk = jnp.dot(ctx, wo_ref[...],
                            preferred_element_type=jnp.float32)
        agbuf[pl.ds(my * CHUNK, CHUNK), :] = out_chunk.astype(jnp.bfloat16)

        for j in range(N_DEV):
            @pl.when(j != my)
            def _():
                pltpu.make_async_remote_copy(
                    src_ref=agbuf.at[pl.ds(my * CHUNK, CHUNK), :],
                    dst_ref=agbuf.at[pl.ds(my * CHUNK, CHUNK), :],
                    send_sem=send_ag.at[j],
                    recv_sem=recv_ag.at[my],
                    device_id=(j,),
                    device_id_type=pl.DeviceIdType.MESH,
                ).start()
        for j in range(N_DEV):
            @pl.when(j != my)
            def _():
                pltpu.make_async_remote_copy(
                    src_ref=agbuf.at[pl.ds(j * CHUNK, CHUNK), :],
                    dst_ref=agbuf.at[pl.ds(j * CHUNK, CHUNK), :],
                    send_sem=recv_ag.at[j], recv_sem=recv_ag.at[j],
                    device_id=(j,), device_id_type=pl.DeviceIdType.MESH,
                ).wait_recv()
        out_ref[0, :, :] = agbuf[...].astype(jnp.float32)

        for j in range(N_DEV):
            pltpu.make_async_remote_copy(
                src_ref=nbuf.at[pl.ds(j * CHUNK, CHUNK), :],
                dst_ref=rs_n.at[my],
                send_sem=send_n.at[j], recv_sem=recv_n.at[my],
                device_id=(j,), device_id_type=pl.DeviceIdType.MESH,
            ).wait_send()
            pltpu.make_async_remote_copy(
                src_ref=mlbuf.at[pl.ds(j * CHUNK, CHUNK), :],
                dst_ref=rs_ml.at[my],
                send_sem=send_ml.at[j], recv_sem=recv_ml.at[my],
                device_id=(j,), device_id_type=pl.DeviceIdType.MESH,
            ).wait_send()
            @pl.when(j != my)
            def _():
                pltpu.make_async_remote_copy(
                    src_ref=agbuf.at[pl.ds(my * CHUNK, CHUNK), :],
                    dst_ref=agbuf.at[pl.ds(my * CHUNK, CHUNK), :],
                    send_sem=send_ag.at[j], recv_sem=recv_ag.at[my],
                    device_id=(j,), device_id_type=pl.DeviceIdType.MESH,
                ).wait_send()

    return pl.pallas_call(
        body,
        out_shape=jax.ShapeDtypeStruct((1, SQ, DM), jnp.float32),
        in_specs=[pl.BlockSpec(memory_space=pltpu.VMEM)] * 5,
        out_specs=pl.BlockSpec(memory_space=pltpu.VMEM),
        scratch_shapes=[
            pltpu.VMEM((SQ, DM), jnp.bfloat16),
            pltpu.VMEM((SQ, ML), jnp.float32),
            pltpu.VMEM((N_DEV, CHUNK, DM), jnp.bfloat16),
            pltpu.VMEM((N_DEV, CHUNK, ML), jnp.float32),
            pltpu.VMEM((SQ, DM), jnp.bfloat16),
            pltpu.SemaphoreType.DMA((N_DEV,)),
            pltpu.SemaphoreType.DMA((N_DEV,)),
            pltpu.SemaphoreType.DMA((N_DEV,)),
            pltpu.SemaphoreType.DMA((N_DEV,)),
            pltpu.SemaphoreType.DMA((N_DEV,)),
            pltpu.SemaphoreType.DMA((N_DEV,)),
        ],
        compiler_params=pltpu.CompilerParams(
            vmem_limit_bytes=128 * 1024 * 1024,
        ),
    )(x, Wq, K_ext, V_ext, Wo)
